# Initial kernel scaffold; baseline (speedup 1.0000x reference)
import jax
import jax.numpy as jnp
from jax import lax
from jax.experimental import pallas as pl
from jax.experimental.pallas import tpu as pltpu
from jax.experimental.pallas import tpu_sc as plsc

N = 10000
E = 160000
D_IN = 128
D_E = 16
D_H = 32
STEPS = 3

NC = 2
NS = 16
NW = NC * NS
EPW = E // NW
G_CHUNK = 1000
S_BATCH = 125
S_ROWS = EPW // S_BATCH
ROWS_PER_TILE = N // NS

_SC_MESH = plsc.VectorSubcoreMesh(core_axis_name="c", subcore_axis_name="s")


def _gather_body(table_hbm, src_hbm, xs_hbm, idx_v, rows_v, sem):
    wid = lax.axis_index("s") * NC + lax.axis_index("c")
    base = wid * EPW
    pltpu.sync_copy(src_hbm.at[pl.ds(base, EPW)], idx_v)
    for c in range(EPW // G_CHUNK):
        pltpu.async_copy(
            table_hbm.at[idx_v.at[pl.ds(c * G_CHUNK, G_CHUNK)]], rows_v, sem
        ).wait()
        pltpu.sync_copy(rows_v, xs_hbm.at[pl.ds(base + c * G_CHUNK, G_CHUNK)])


_gather = pl.kernel(
    _gather_body,
    out_type=jax.ShapeDtypeStruct((E, D_H), jnp.float32),
    mesh=_SC_MESH,
    scratch_types=[
        pltpu.VMEM((EPW,), jnp.int32),
        pltpu.VMEM((G_CHUNK, D_H), jnp.float32),
        pltpu.SemaphoreType.DMA,
    ],
)


def _scatter_body(msg_hbm, dst_hbm, zeros_hbm, parts_hbm, idx_v, rows_v, shared):
    cid = lax.axis_index("c")
    sid = lax.axis_index("s")
    wid = sid * NC + cid
    base = wid * EPW
    row0 = sid * ROWS_PER_TILE
    pltpu.sync_copy(
        zeros_hbm.at[pl.ds(row0, ROWS_PER_TILE)],
        shared.at[pl.ds(row0, ROWS_PER_TILE)],
    )
    pltpu.sync_copy(dst_hbm.at[wid], idx_v)
    plsc.subcore_barrier()
    half = EPW // 2
    for hlf in range(2):
        pltpu.sync_copy(msg_hbm.at[pl.ds(base + hlf * half, half)], rows_v)
        for c in range(half // S_BATCH):
            r = hlf * (half // S_BATCH) + c
            pltpu.sync_copy(
                rows_v.at[pl.ds(c * S_BATCH, S_BATCH)],
                shared.at[idx_v.at[r]],
                add=True,
            )
    plsc.subcore_barrier()
    pltpu.sync_copy(
        shared.at[pl.ds(row0, ROWS_PER_TILE)],
        parts_hbm.at[cid, pl.ds(row0, ROWS_PER_TILE)],
    )


_scatter = pl.kernel(
    _scatter_body,
    out_type=jax.ShapeDtypeStruct((NC, N, D_H), jnp.float32),
    mesh=_SC_MESH,
    scratch_types=[
        pltpu.VMEM((S_ROWS, S_BATCH), jnp.int32),
        pltpu.VMEM((EPW // 2, D_H), jnp.float32),
        pltpu.VMEM_SHARED((N, D_H), jnp.float32),
    ],
)


def _linrelu_body(x_ref, wt_ref, b_ref, o_ref):
    o_ref[...] = jax.nn.relu(
        jnp.dot(x_ref[...], wt_ref[...], preferred_element_type=jnp.float32)
        + b_ref[...]
    )


def _linrelu(x, w, b, tile):
    rows, din = x.shape
    dout = w.shape[0]
    grid = rows // tile
    return pl.pallas_call(
        _linrelu_body,
        grid=(grid,),
        in_specs=[
            pl.BlockSpec((tile, din), lambda i: (i, 0)),
            pl.BlockSpec((din, dout), lambda i: (0, 0)),
            pl.BlockSpec((1, dout), lambda i: (0, 0)),
        ],
        out_specs=pl.BlockSpec((tile, dout), lambda i: (i, 0)),
        out_shape=jax.ShapeDtypeStruct((rows, dout), jnp.float32),
    )(x, w.T, b.reshape(1, dout))


TE = 2000


def _bilinear_body(xs_ref, g_ref, a2_ref, bmat_ref, o_ref):
    xs = xs_ref[...]
    g = g_ref[...]
    c = jnp.dot(xs, a2_ref[...], preferred_element_type=jnp.float32)
    msg = jnp.dot(xs, bmat_ref[...], preferred_element_type=jnp.float32)
    for h in range(D_H):
        msg = msg + g[:, h : h + 1] * c[:, h * D_H : (h + 1) * D_H]
    o_ref[...] = msg


def _bilinear(xs, g, a2, bmat):
    return pl.pallas_call(
        _bilinear_body,
        grid=(E // TE,),
        in_specs=[
            pl.BlockSpec((TE, D_H), lambda i: (i, 0)),
            pl.BlockSpec((TE, D_H), lambda i: (i, 0)),
            pl.BlockSpec((D_H, D_H * D_H), lambda i: (0, 0)),
            pl.BlockSpec((D_H, D_H), lambda i: (0, 0)),
        ],
        out_specs=pl.BlockSpec((TE, D_H), lambda i: (i, 0)),
        out_shape=jax.ShapeDtypeStruct((E, D_H), jnp.float32),
    )(xs, g, a2, bmat)


TN = 2000


def _gru_body(p0_ref, p1_ref, h_ref, wih_ref, whh_ref, bih_ref, bhh_ref, o_ref):
    m = jax.nn.relu(p0_ref[...] + p1_ref[...])
    h = h_ref[...]
    gi = jnp.dot(m, wih_ref[...], preferred_element_type=jnp.float32) + bih_ref[...]
    gh = jnp.dot(h, whh_ref[...], preferred_element_type=jnp.float32) + bhh_ref[...]
    r = jax.nn.sigmoid(gi[:, :D_H] + gh[:, :D_H])
    z = jax.nn.sigmoid(gi[:, D_H : 2 * D_H] + gh[:, D_H : 2 * D_H])
    n = jnp.tanh(gi[:, 2 * D_H :] + r * gh[:, 2 * D_H :])
    o_ref[...] = (1.0 - z) * n + z * h


def _gru(parts, h, wih_t, whh_t, bih, bhh):
    pflat = parts.reshape(NC * N, D_H)
    nb = N // TN
    return pl.pallas_call(
        _gru_body,
        grid=(nb,),
        in_specs=[
            pl.BlockSpec((TN, D_H), lambda i: (i, 0)),
            pl.BlockSpec((TN, D_H), lambda i, _nb=nb: (_nb + i, 0)),
            pl.BlockSpec((TN, D_H), lambda i: (i, 0)),
            pl.BlockSpec((D_H, 3 * D_H), lambda i: (0, 0)),
            pl.BlockSpec((D_H, 3 * D_H), lambda i: (0, 0)),
            pl.BlockSpec((1, 3 * D_H), lambda i: (0, 0)),
            pl.BlockSpec((1, 3 * D_H), lambda i: (0, 0)),
        ],
        out_specs=pl.BlockSpec((TN, D_H), lambda i: (i, 0)),
        out_shape=jax.ShapeDtypeStruct((N, D_H), jnp.float32),
    )(pflat, pflat, h, wih_t, whh_t, bih, bhh)


def kernel(n_feat, edge_index, e_feat, lin0_w, lin0_b, en1_w, en1_b, en2_w,
           en2_b, gru_wih, gru_whh, gru_bih, gru_bhh):
    src = edge_index[0]
    dst = edge_index[1].reshape(NW, S_ROWS, S_BATCH)
    zeros = jnp.zeros((N, D_H), jnp.float32)

    a2 = en2_w.reshape(D_H, D_H, D_H).transpose(0, 2, 1).reshape(D_H, D_H * D_H)
    bmat = en2_b.reshape(D_H, D_H)
    wih_t = gru_wih.T
    whh_t = gru_whh.T
    bih = gru_bih.reshape(1, 3 * D_H)
    bhh = gru_bhh.reshape(1, 3 * D_H)

    out = _linrelu(n_feat, lin0_w, lin0_b, tile=2000)
    g = _linrelu(e_feat, en1_w, en1_b, tile=8000)
    h = out
    for _ in range(STEPS):
        xs = _gather(out, src)
        msg = _bilinear(xs, g, a2, bmat)
        parts = _scatter(msg, dst, zeros)
        out = _gru(parts, h, wih_t, whh_t, bih, bhh)
        h = out
    return out


# speedup vs baseline: 1.0878x; 1.0878x over previous
import functools

import jax
import jax.numpy as jnp
from jax import lax
from jax.experimental import pallas as pl
from jax.experimental.pallas import tpu as pltpu
from jax.experimental.pallas import tpu_sc as plsc

N = 10000
E = 160000
D_IN = 128
D_E = 16
D_H = 32
STEPS = 3

NC = 2
NS = 16
NW = NC * NS
EPW = E // NW
G_CHUNK = 1000
S_BATCH = 125
S_ROWS = EPW // S_BATCH
ROWS_PER_TILE = N // NS

def _gather_body(table_hbm, src_hbm, xs_hbm, idx_v, rows_v, sem):
    wid = lax.axis_index("s") * NC + lax.axis_index("c")
    base = wid * EPW
    pltpu.sync_copy(src_hbm.at[pl.ds(base, EPW)], idx_v)
    for c in range(EPW // G_CHUNK):
        pltpu.async_copy(
            table_hbm.at[idx_v.at[pl.ds(c * G_CHUNK, G_CHUNK)]], rows_v, sem
        ).wait()
        pltpu.sync_copy(rows_v, xs_hbm.at[pl.ds(base + c * G_CHUNK, G_CHUNK)])


@functools.cache
def _get_gather():
    return pl.kernel(
        _gather_body,
        out_type=jax.ShapeDtypeStruct((E, D_H), jnp.float32),
        mesh=plsc.VectorSubcoreMesh(core_axis_name="c", subcore_axis_name="s"),
        scratch_types=[
            pltpu.VMEM((EPW,), jnp.int32),
            pltpu.VMEM((G_CHUNK, D_H), jnp.float32),
            pltpu.SemaphoreType.DMA,
        ],
        compiler_params=pltpu.CompilerParams(use_tc_tiling_on_sc=False),
    )


def _gather(table, src):
    return _get_gather()(table, src)


def _scatter_body(msg_hbm, dst_hbm, zeros_hbm, parts_hbm, idx_v, rows_v, shared):
    cid = lax.axis_index("c")
    sid = lax.axis_index("s")
    wid = sid * NC + cid
    base = wid * EPW
    row0 = sid * ROWS_PER_TILE
    pltpu.sync_copy(
        zeros_hbm.at[pl.ds(row0, ROWS_PER_TILE)],
        shared.at[pl.ds(row0, ROWS_PER_TILE)],
    )
    pltpu.sync_copy(dst_hbm.at[wid], idx_v)
    plsc.subcore_barrier()
    half = EPW // 2
    for hlf in range(2):
        pltpu.sync_copy(msg_hbm.at[pl.ds(base + hlf * half, half)], rows_v)
        for c in range(half // S_BATCH):
            r = hlf * (half // S_BATCH) + c
            pltpu.sync_copy(
                rows_v.at[pl.ds(c * S_BATCH, S_BATCH)],
                shared.at[idx_v.at[r]],
                add=True,
            )
    plsc.subcore_barrier()
    pltpu.sync_copy(
        shared.at[pl.ds(row0, ROWS_PER_TILE)],
        parts_hbm.at[cid, pl.ds(row0, ROWS_PER_TILE)],
    )


@functools.cache
def _get_scatter():
    return pl.kernel(
        _scatter_body,
        out_type=jax.ShapeDtypeStruct((NC, N, D_H), jnp.float32),
        mesh=plsc.VectorSubcoreMesh(core_axis_name="c", subcore_axis_name="s"),
        scratch_types=[
            pltpu.VMEM((S_ROWS, S_BATCH), jnp.int32),
            pltpu.VMEM((EPW // 2, D_H), jnp.float32),
            pltpu.VMEM_SHARED((N, D_H), jnp.float32),
        ],
        compiler_params=pltpu.CompilerParams(use_tc_tiling_on_sc=False),
    )


def _scatter(msg, dst3, zeros):
    return _get_scatter()(msg, dst3, zeros)


def _linrelu_body(x_ref, wt_ref, b_ref, o_ref):
    o_ref[...] = jax.nn.relu(
        jnp.dot(x_ref[...], wt_ref[...], preferred_element_type=jnp.float32)
        + b_ref[...]
    ).astype(o_ref.dtype)


def _linrelu(x, w, b, tile, out_dtype=jnp.float32):
    rows, din = x.shape
    dout = w.shape[0]
    grid = rows // tile
    return pl.pallas_call(
        _linrelu_body,
        grid=(grid,),
        in_specs=[
            pl.BlockSpec((tile, din), lambda i: (i, 0)),
            pl.BlockSpec((din, dout), lambda i: (0, 0)),
            pl.BlockSpec((1, dout), lambda i: (0, 0)),
        ],
        out_specs=pl.BlockSpec((tile, dout), lambda i: (i, 0)),
        out_shape=jax.ShapeDtypeStruct((rows, dout), out_dtype),
    )(x, w.T, b.reshape(1, dout))


TE = 1000


def _bilinear_body(xs_ref, g_ref, a2t_ref, b2_ref, o_ref):
    xs = xs_ref[...].astype(jnp.bfloat16).astype(jnp.float32)
    d = jnp.dot(g_ref[...], a2t_ref[...], preferred_element_type=jnp.float32)
    d = (d + b2_ref[...]).astype(jnp.bfloat16)
    acc = xs[:, 0:1] * d[:, 0:D_H].astype(jnp.float32)
    for i in range(1, D_H):
        acc = acc + xs[:, i : i + 1] * d[:, i * D_H : (i + 1) * D_H].astype(
            jnp.float32
        )
    o_ref[...] = acc


def _bilinear(xs, g, a2t, b2):
    return pl.pallas_call(
        _bilinear_body,
        grid=(E // TE,),
        in_specs=[
            pl.BlockSpec((TE, D_H), lambda i: (i, 0)),
            pl.BlockSpec((TE, D_H), lambda i: (i, 0)),
            pl.BlockSpec((D_H, D_H * D_H), lambda i: (0, 0)),
            pl.BlockSpec((1, D_H * D_H), lambda i: (0, 0)),
        ],
        out_specs=pl.BlockSpec((TE, D_H), lambda i: (i, 0)),
        out_shape=jax.ShapeDtypeStruct((E, D_H), jnp.float32),
    )(xs, g, a2t, b2)


TN = 2000


def _gru_body(p0_ref, p1_ref, h_ref, wih_ref, whh_ref, bih_ref, bhh_ref, o_ref):
    m = jax.nn.relu(p0_ref[...] + p1_ref[...])
    h = h_ref[...]
    gi = jnp.dot(m, wih_ref[...], preferred_element_type=jnp.float32) + bih_ref[...]
    gh = jnp.dot(h, whh_ref[...], preferred_element_type=jnp.float32) + bhh_ref[...]
    r = jax.nn.sigmoid(gi[:, :D_H] + gh[:, :D_H])
    z = jax.nn.sigmoid(gi[:, D_H : 2 * D_H] + gh[:, D_H : 2 * D_H])
    n = jnp.tanh(gi[:, 2 * D_H :] + r * gh[:, 2 * D_H :])
    o_ref[...] = (1.0 - z) * n + z * h


def _gru(parts, h, wih_t, whh_t, bih, bhh):
    pflat = parts.reshape(NC * N, D_H)
    nb = N // TN
    return pl.pallas_call(
        _gru_body,
        grid=(nb,),
        in_specs=[
            pl.BlockSpec((TN, D_H), lambda i: (i, 0)),
            pl.BlockSpec((TN, D_H), lambda i, _nb=nb: (_nb + i, 0)),
            pl.BlockSpec((TN, D_H), lambda i: (i, 0)),
            pl.BlockSpec((D_H, 3 * D_H), lambda i: (0, 0)),
            pl.BlockSpec((D_H, 3 * D_H), lambda i: (0, 0)),
            pl.BlockSpec((1, 3 * D_H), lambda i: (0, 0)),
            pl.BlockSpec((1, 3 * D_H), lambda i: (0, 0)),
        ],
        out_specs=pl.BlockSpec((TN, D_H), lambda i: (i, 0)),
        out_shape=jax.ShapeDtypeStruct((N, D_H), jnp.float32),
    )(pflat, pflat, h, wih_t, whh_t, bih, bhh)


def kernel(n_feat, edge_index, e_feat, lin0_w, lin0_b, en1_w, en1_b, en2_w,
           en2_b, gru_wih, gru_whh, gru_bih, gru_bhh):
    src = edge_index[0]
    dst = edge_index[1].reshape(NW, S_ROWS, S_BATCH)
    zeros = jnp.zeros((N, D_H), jnp.float32)

    a2t = en2_w.T.astype(jnp.bfloat16)
    b2 = en2_b.reshape(1, D_H * D_H)
    wih_t = gru_wih.T
    whh_t = gru_whh.T
    bih = gru_bih.reshape(1, 3 * D_H)
    bhh = gru_bhh.reshape(1, 3 * D_H)

    out = _linrelu(n_feat, lin0_w, lin0_b, tile=2000)
    g = _linrelu(e_feat, en1_w, en1_b, tile=4000, out_dtype=jnp.bfloat16)
    h = out
    for _ in range(STEPS):
        xs = _gather(out, src)
        msg = _bilinear(xs, g, a2t, b2)
        parts = _scatter(msg, dst, zeros)
        out = _gru(parts, h, wih_t, whh_t, bih, bhh)
        h = out
    return out
